# Initial kernel scaffold; baseline (speedup 1.0000x reference)
import jax
import jax.numpy as jnp
from jax.experimental import pallas as pl


def kernel(indices, table):
    raise NotImplementedError("write your pallas kernel here")


# speedup vs baseline: 4.1387x; 4.1387x over previous
import functools

import jax
import jax.numpy as jnp
from jax import lax
from jax.experimental import pallas as pl
from jax.experimental.pallas import tpu as pltpu
from jax.experimental.pallas import tpu_sc as plsc

_NC = 2
_NS = 16
_NW = _NC * _NS


@functools.lru_cache(maxsize=None)
def _make_gather(n_rows: int, d: int, chunk: int):
    b_per_w = n_rows // _NW
    n_groups = b_per_w // chunk
    k = chunk // 128

    mesh = plsc.VectorSubcoreMesh(core_axis_name="c", subcore_axis_name="s")

    @functools.partial(
        pl.kernel,
        mesh=mesh,
        compiler_params=pltpu.CompilerParams(use_tc_tiling_on_sc=False),
        out_type=jax.ShapeDtypeStruct((n_rows, d), jnp.float32),
        scratch_types=[
            pltpu.VMEM((k, 128), jnp.int32),
            pltpu.VMEM((chunk, d), jnp.float32),
            pltpu.SemaphoreType.DMA,
        ],
    )
    def gather_kernel(idx_hbm, table_hbm, out_hbm, idx_v, rows_v, sem):
        wid = lax.axis_index("s") * _NC + lax.axis_index("c")
        base = wid * b_per_w

        def body(g, _):
            row0 = pl.multiple_of(base + g * chunk, chunk)
            irow0 = pl.multiple_of(wid * (b_per_w // 128) + g * k, 8)
            pltpu.sync_copy(idx_hbm.at[pl.ds(irow0, k)], idx_v)
            copies = [
                pltpu.async_copy(
                    table_hbm.at[idx_v.at[j]],
                    rows_v.at[pl.ds(j * 128, 128)],
                    sem,
                )
                for j in range(k)
            ]
            for c in copies:
                c.wait()
            pltpu.sync_copy(rows_v, out_hbm.at[pl.ds(row0, chunk)])
            return _

        lax.fori_loop(0, n_groups, body, 0)

    return gather_kernel


def kernel(indices, table):
    batch, hist = indices.shape
    d = table.shape[1]
    flat = indices.reshape(-1).astype(jnp.int32)
    n_rows = flat.shape[0]
    idx2d = flat.reshape(n_rows // 128, 128)
    out = _make_gather(n_rows, d, 1024)(idx2d, table)
    return out.reshape(batch, hist, d)


# speedup vs baseline: 4.2614x; 1.0297x over previous
import functools

import jax
import jax.numpy as jnp
from jax import lax
from jax.experimental import pallas as pl
from jax.experimental.pallas import tpu as pltpu
from jax.experimental.pallas import tpu_sc as plsc

_NC = 2
_NS = 16
_NW = _NC * _NS


@functools.lru_cache(maxsize=None)
def _make_gather(n_rows: int, d: int, chunk: int):
    b_per_w = n_rows // _NW
    n_groups = b_per_w // chunk
    assert n_groups % 2 == 0 and chunk % 128 == 0
    n_pairs = n_groups // 2
    k = chunk // 128
    idx_rows = b_per_w // 128

    mesh = plsc.VectorSubcoreMesh(core_axis_name="c", subcore_axis_name="s")

    @functools.partial(
        pl.kernel,
        mesh=mesh,
        compiler_params=pltpu.CompilerParams(use_tc_tiling_on_sc=False),
        out_type=jax.ShapeDtypeStruct((n_rows, d), jnp.float32),
        scratch_types=[
            pltpu.VMEM((idx_rows, 128), jnp.int32),
            pltpu.VMEM((chunk, d), jnp.float32),
            pltpu.VMEM((chunk, d), jnp.float32),
            pltpu.SemaphoreType.DMA,
            pltpu.SemaphoreType.DMA,
            pltpu.SemaphoreType.DMA,
            pltpu.SemaphoreType.DMA,
        ],
    )
    def gather_kernel(idx_hbm, table_hbm, out_hbm, idx_v, rows0, rows1,
                      gsem0, gsem1, osem0, osem1):
        wid = lax.axis_index("s") * _NC + lax.axis_index("c")
        base = wid * b_per_w

        def fire_gather(g, rows_v, gsem):
            for j in range(k):
                pltpu.async_copy(
                    table_hbm.at[idx_v.at[g * k + j]],
                    rows_v.at[pl.ds(j * 128, 128)],
                    gsem,
                )

        def wait_gather(g, rows_v, gsem):
            for j in range(k):
                pltpu.make_async_copy(
                    table_hbm.at[idx_v.at[g * k + j]],
                    rows_v.at[pl.ds(j * 128, 128)],
                    gsem,
                ).wait()

        def fire_out(g, rows_v, osem):
            pltpu.async_copy(rows_v, out_hbm.at[pl.ds(base + g * chunk, chunk)], osem)

        def wait_out(g, rows_v, osem):
            pltpu.make_async_copy(
                rows_v, out_hbm.at[pl.ds(base + g * chunk, chunk)], osem
            ).wait()

        pltpu.sync_copy(idx_hbm.at[pl.ds(wid * idx_rows, idx_rows)], idx_v)
        fire_gather(0, rows0, gsem0)

        def body(t, carry):
            g0 = 2 * t
            g1 = g0 + 1

            @pl.when(t >= 1)
            def _wait_prev_out1():
                wait_out(g1 - 2, rows1, osem1)

            fire_gather(g1, rows1, gsem1)
            wait_gather(g0, rows0, gsem0)
            fire_out(g0, rows0, osem0)

            @pl.when(t < n_pairs - 1)
            def _refill_buf0():
                wait_out(g0, rows0, osem0)
                fire_gather(g0 + 2, rows0, gsem0)

            wait_gather(g1, rows1, gsem1)
            fire_out(g1, rows1, osem1)
            return carry

        lax.fori_loop(0, n_pairs, body, 0)
        wait_out(n_groups - 2, rows0, osem0)
        wait_out(n_groups - 1, rows1, osem1)

    return gather_kernel


def kernel(indices, table):
    batch, hist = indices.shape
    d = table.shape[1]
    flat = indices.reshape(-1).astype(jnp.int32)
    n_rows = flat.shape[0]
    idx2d = flat.reshape(n_rows // 128, 128)
    out = _make_gather(n_rows, d, 640)(idx2d, table)
    return out.reshape(batch, hist, d)


# speedup vs baseline: 4.2634x; 1.0005x over previous
import functools

import jax
import jax.numpy as jnp
from jax import lax
from jax.experimental import pallas as pl
from jax.experimental.pallas import tpu as pltpu
from jax.experimental.pallas import tpu_sc as plsc

_NC = 2
_NS = 16
_NW = _NC * _NS


@functools.lru_cache(maxsize=None)
def _make_gather(n_rows: int, d: int, chunk: int):
    b_per_w = n_rows // _NW
    n_groups = b_per_w // chunk
    assert n_groups % 2 == 0 and chunk % 128 == 0
    n_pairs = n_groups // 2
    k = chunk // 128
    idx_rows = b_per_w // 128

    mesh = plsc.VectorSubcoreMesh(core_axis_name="c", subcore_axis_name="s")

    @functools.partial(
        pl.kernel,
        mesh=mesh,
        compiler_params=pltpu.CompilerParams(use_tc_tiling_on_sc=False),
        out_type=jax.ShapeDtypeStruct((n_rows, d), jnp.float32),
        scratch_types=[
            pltpu.VMEM((b_per_w,), jnp.int32),
            pltpu.VMEM((chunk, d), jnp.float32),
            pltpu.VMEM((chunk, d), jnp.float32),
            pltpu.SemaphoreType.DMA,
            pltpu.SemaphoreType.DMA,
            pltpu.SemaphoreType.DMA,
            pltpu.SemaphoreType.DMA,
        ],
    )
    def gather_kernel(idx_hbm, table_hbm, out_hbm, idx_v, rows0, rows1,
                      gsem0, gsem1, osem0, osem1):
        wid = lax.axis_index("s") * _NC + lax.axis_index("c")
        base = wid * b_per_w

        def fire_gather(g, rows_v, gsem):
            pltpu.async_copy(
                table_hbm.at[idx_v.at[pl.ds(g * chunk, chunk)]],
                rows_v,
                gsem,
            )

        def wait_gather(g, rows_v, gsem):
            pltpu.make_async_copy(
                table_hbm.at[idx_v.at[pl.ds(g * chunk, chunk)]],
                rows_v,
                gsem,
            ).wait()

        def fire_out(g, rows_v, osem):
            pltpu.async_copy(rows_v, out_hbm.at[pl.ds(base + g * chunk, chunk)], osem)

        def wait_out(g, rows_v, osem):
            pltpu.make_async_copy(
                rows_v, out_hbm.at[pl.ds(base + g * chunk, chunk)], osem
            ).wait()

        pltpu.sync_copy(idx_hbm.at[pl.ds(base, b_per_w)], idx_v)
        fire_gather(0, rows0, gsem0)

        def body(t, carry):
            g0 = 2 * t
            g1 = g0 + 1

            @pl.when(t >= 1)
            def _wait_prev_out1():
                wait_out(g1 - 2, rows1, osem1)

            fire_gather(g1, rows1, gsem1)
            wait_gather(g0, rows0, gsem0)
            fire_out(g0, rows0, osem0)

            @pl.when(t < n_pairs - 1)
            def _refill_buf0():
                wait_out(g0, rows0, osem0)
                fire_gather(g0 + 2, rows0, gsem0)

            wait_gather(g1, rows1, gsem1)
            fire_out(g1, rows1, osem1)
            return carry

        lax.fori_loop(0, n_pairs, body, 0)
        wait_out(n_groups - 2, rows0, osem0)
        wait_out(n_groups - 1, rows1, osem1)

    return gather_kernel


def kernel(indices, table):
    batch, hist = indices.shape
    d = table.shape[1]
    flat = indices.reshape(-1).astype(jnp.int32)
    n_rows = flat.shape[0]
    out = _make_gather(n_rows, d, 640)(flat, table)
    return out.reshape(batch, hist, d)
